# baseline (speedup 1.0000x reference)
import jax
import jax.numpy as jnp
from jax.experimental import pallas as pl

SEQ_LEN = 8192


def _gate_body(gs_ref, out_ref):
    g0 = gs_ref[0:1, :]
    g1 = gs_ref[1:2, :]
    m = jnp.maximum(g0, g1)
    e0 = jnp.exp(g0 - m)
    e1 = jnp.exp(g1 - m)
    denom = e0 + e1
    s0 = e0 / denom
    s1 = e1 / denom
    sel = g0 >= g1
    out_ref[0, :, :] = jnp.where(sel, 1.0, 0.0) - s0 + s0
    out_ref[1, :, :] = jnp.where(sel, 0.0, 1.0) - s1 + s1


def kernel(x, W, global_gate_score, smoothing_factor):
    del x, W, smoothing_factor
    gt = global_gate_score.T
    res = pl.pallas_call(
        _gate_body,
        out_shape=jax.ShapeDtypeStruct((2, 1, SEQ_LEN), jnp.float32),
    )(gt)
    return res[0, 0], res[1, 0]


# speedup vs baseline: 1.7229x; 1.7229x over previous
import jax
import jax.numpy as jnp
from jax.experimental import pallas as pl
from jax.experimental.pallas import tpu as pltpu

SEQ_LEN = 8192


def _gate_body(gs_hbm, out0_ref, out1_ref, buf, sem):
    copy = pltpu.make_async_copy(gs_hbm, buf, sem)
    copy.start()
    copy.wait()
    g0 = buf[0:1, :]
    g1 = buf[1:2, :]
    m = jnp.maximum(g0, g1)
    e0 = jnp.exp(g0 - m)
    e1 = jnp.exp(g1 - m)
    denom = e0 + e1
    s0 = e0 / denom
    s1 = e1 / denom
    sel = g0 >= g1
    out0_ref[...] = jnp.where(sel, 1.0, 0.0) - s0 + s0
    out1_ref[...] = jnp.where(sel, 0.0, 1.0) - s1 + s1


def kernel(x, W, global_gate_score, smoothing_factor):
    del x, W, smoothing_factor
    gt = global_gate_score.T
    out0, out1 = pl.pallas_call(
        _gate_body,
        in_specs=[pl.BlockSpec(memory_space=pl.ANY)],
        out_shape=(
            jax.ShapeDtypeStruct((1, SEQ_LEN), jnp.float32),
            jax.ShapeDtypeStruct((1, SEQ_LEN), jnp.float32),
        ),
        scratch_shapes=[
            pltpu.VMEM((2, SEQ_LEN), jnp.float32),
            pltpu.SemaphoreType.DMA,
        ],
    )(gt)
    return out0.reshape(SEQ_LEN), out1.reshape(SEQ_LEN)
